# baseline (speedup 1.0000x reference)
import functools

import jax
import jax.numpy as jnp
from jax import lax
from jax.experimental import pallas as pl
from jax.experimental.pallas import tpu as pltpu
from jax.experimental.pallas import tpu_sc as plsc

PORT_DIM = 32
PROTO_DIM = 8
PACK_DIM = PORT_DIM + PROTO_DIM
LANES = 16


def _sc_gather_t(table4, pidx, ptable_t, qidx, *, nc, ns, b):
    nw = nc * ns
    vhi, vlo = table4.shape[1], table4.shape[3]
    pv = ptable_t.shape[1]
    qchunk = b // (nw // PROTO_DIM)
    mesh = plsc.VectorSubcoreMesh(core_axis_name="c", subcore_axis_name="s")

    @functools.partial(
        pl.kernel,
        mesh=mesh,
        compiler_params=pltpu.CompilerParams(use_tc_tiling_on_sc=False,
                                             needs_layout_passes=False),
        out_type=jax.ShapeDtypeStruct((PACK_DIM, b), jnp.float32),
        scratch_types=[
            pltpu.VMEM((vhi, vlo), jnp.float32),
            pltpu.VMEM((pv,), jnp.float32),
            pltpu.VMEM((b,), jnp.int32),
            pltpu.VMEM((qchunk,), jnp.int32),
            pltpu.VMEM((b,), jnp.float32),
            pltpu.VMEM((qchunk,), jnp.float32),
            pltpu.SemaphoreType.DMA,
        ],
    )
    def gather(tbl, pidx_hbm, ptbl, qidx_hbm, out,
               row_v, prow_v, pidx_v, qidx_v, pout_v, qout_v, sem):
        wid = lax.axis_index("s") * nc + lax.axis_index("c")
        qf = wid % PROTO_DIM
        qb = (wid // PROTO_DIM) * qchunk
        copies = [
            pltpu.async_copy(tbl.at[wid // 8, :, wid % 8, :], row_v, sem),
            pltpu.async_copy(ptbl.at[qf], prow_v, sem),
            pltpu.async_copy(pidx_hbm, pidx_v, sem),
            pltpu.async_copy(qidx_hbm.at[pl.ds(qb, qchunk)], qidx_v, sem),
        ]
        for c in copies:
            c.wait()

        @plsc.parallel_loop(0, b // LANES, unroll=16)
        def port_body(i):
            vec = pidx_v[pl.ds(i * LANES, LANES)]
            pout_v[pl.ds(i * LANES, LANES)] = plsc.load_gather(
                row_v, [lax.shift_right_logical(vec, 7),
                        lax.bitwise_and(vec, 127)])

        @plsc.parallel_loop(0, qchunk // LANES, unroll=16)
        def proto_body(i):
            vec = qidx_v[pl.ds(i * LANES, LANES)]
            qout_v[pl.ds(i * LANES, LANES)] = plsc.load_gather(prow_v, [vec])

        pltpu.sync_copy(pout_v, out.at[wid])
        pltpu.sync_copy(qout_v, out.at[PORT_DIM + qf, pl.ds(qb, qchunk)])

    return gather(table4, pidx, ptable_t, qidx)


def _mlp_body(packed, feat, w1ab, w1c, b1, w2, b2, w3t, b3, out):
    h = jnp.dot(feat[...], w1c[...], preferred_element_type=jnp.float32)
    h += lax.dot_general(packed[...], w1ab[...],
                         (((0,), (0,)), ((), ())),
                         preferred_element_type=jnp.float32)
    h = jnp.maximum(h + b1[...], 0.0)
    h = jnp.dot(h, w2[...], preferred_element_type=jnp.float32) + b2[...]
    h = jnp.maximum(h, 0.0)
    out[...] = lax.dot_general(w3t[...], h, (((1,), (1,)), ((), ())),
                               preferred_element_type=jnp.float32) + b3[...]


def _mlp(packed_t, features, w1ab, w1c, b1, w2, b2, w3t, b3, *, bm):
    b = features.shape[0]
    n_out = w3t.shape[0]
    const = lambda i: (0, 0)
    return pl.pallas_call(
        _mlp_body,
        grid=(b // bm,),
        in_specs=[
            pl.BlockSpec((PACK_DIM, bm), lambda i: (0, i)),
            pl.BlockSpec((bm, features.shape[1]), lambda i: (i, 0)),
            pl.BlockSpec(w1ab.shape, const),
            pl.BlockSpec(w1c.shape, const),
            pl.BlockSpec(b1.shape, const),
            pl.BlockSpec(w2.shape, const),
            pl.BlockSpec(b2.shape, const),
            pl.BlockSpec(w3t.shape, const),
            pl.BlockSpec(b3.shape, const),
        ],
        out_specs=pl.BlockSpec((n_out, bm), lambda i: (0, i)),
        out_shape=jax.ShapeDtypeStruct((n_out, b), jnp.float32),
        compiler_params=pltpu.CompilerParams(
            dimension_semantics=("parallel",)),
    )(packed_t, features, w1ab, w1c, b1, w2, b2, w3t, b3)


def kernel(port_idx, protocol_idx, features, port_table, proto_table,
           W1, b1, W2, b2, W3, b3):
    b = port_idx.shape[0]
    info = plsc.get_sparse_core_info()
    nc, ns = info.num_cores, info.num_subcores

    table4 = port_table.T.reshape(4, 8, 512, 128).transpose(0, 2, 1, 3)
    packed_t = _sc_gather_t(table4, port_idx.astype(jnp.int32),
                            proto_table.T, protocol_idx.astype(jnp.int32),
                            nc=nc, ns=ns, b=b)

    out_t = _mlp(packed_t, features, W1[:PACK_DIM], W1[PACK_DIM:],
                 b1.reshape(1, -1), W2, b2.reshape(1, -1),
                 W3.T, b3.reshape(-1, 1), bm=4096)
    return out_t.T


# speedup vs baseline: 1.0132x; 1.0132x over previous
import functools

import jax
import jax.numpy as jnp
from jax import lax
from jax.experimental import pallas as pl
from jax.experimental.pallas import tpu as pltpu
from jax.experimental.pallas import tpu_sc as plsc

PORT_DIM = 32
PROTO_DIM = 8
PACK_DIM = PORT_DIM + PROTO_DIM
LANES = 16


def _sc_gather_t(table4, pidx, ptable_t, qidx, *, nc, ns, b):
    nw = nc * ns
    vhi, vlo = table4.shape[1], table4.shape[3]
    pv = ptable_t.shape[1]
    qchunk = b // (nw // PROTO_DIM)
    mesh = plsc.VectorSubcoreMesh(core_axis_name="c", subcore_axis_name="s")

    @functools.partial(
        pl.kernel,
        mesh=mesh,
        compiler_params=pltpu.CompilerParams(use_tc_tiling_on_sc=False,
                                             needs_layout_passes=False),
        out_type=jax.ShapeDtypeStruct((PACK_DIM, b), jnp.float32),
        scratch_types=[
            pltpu.VMEM((vhi, vlo), jnp.float32),
            pltpu.VMEM((pv,), jnp.float32),
            pltpu.VMEM((b,), jnp.int32),
            pltpu.VMEM((qchunk,), jnp.int32),
            pltpu.VMEM((b,), jnp.float32),
            pltpu.VMEM((qchunk,), jnp.float32),
            pltpu.SemaphoreType.DMA,
        ],
    )
    def gather(tbl, pidx_hbm, ptbl, qidx_hbm, out,
               row_v, prow_v, pidx_v, qidx_v, pout_v, qout_v, sem):
        wid = lax.axis_index("s") * nc + lax.axis_index("c")
        qf = wid % PROTO_DIM
        qb = (wid // PROTO_DIM) * qchunk
        copies = [
            pltpu.async_copy(tbl.at[wid // 8, :, wid % 8, :], row_v, sem),
            pltpu.async_copy(ptbl.at[qf], prow_v, sem),
            pltpu.async_copy(pidx_hbm, pidx_v, sem),
            pltpu.async_copy(qidx_hbm.at[pl.ds(qb, qchunk)], qidx_v, sem),
        ]
        for c in copies:
            c.wait()

        @plsc.parallel_loop(0, b // LANES, unroll=16)
        def port_body(i):
            vec = pidx_v[pl.ds(i * LANES, LANES)]
            pout_v[pl.ds(i * LANES, LANES)] = plsc.load_gather(
                row_v, [lax.shift_right_logical(vec, 7),
                        lax.bitwise_and(vec, 127)])

        @plsc.parallel_loop(0, qchunk // LANES, unroll=16)
        def proto_body(i):
            vec = qidx_v[pl.ds(i * LANES, LANES)]
            qout_v[pl.ds(i * LANES, LANES)] = plsc.load_gather(prow_v, [vec])

        pltpu.sync_copy(pout_v, out.at[wid])
        pltpu.sync_copy(qout_v, out.at[PORT_DIM + qf, pl.ds(qb, qchunk)])

    return gather(table4, pidx, ptable_t, qidx)


def _mlp_body(packed, feat, w1ab, w1c, b1, w2, b2, w3t, b3, out):
    h = jnp.dot(feat[...], w1c[...], preferred_element_type=jnp.float32)
    h += lax.dot_general(packed[...], w1ab[...],
                         (((0,), (0,)), ((), ())),
                         preferred_element_type=jnp.float32)
    h = jnp.maximum(h + b1[...], 0.0)
    h = jnp.dot(h, w2[...], preferred_element_type=jnp.float32) + b2[...]
    h = jnp.maximum(h, 0.0)
    out[...] = lax.dot_general(w3t[...], h, (((1,), (1,)), ((), ())),
                               preferred_element_type=jnp.float32) + b3[...]


def _mlp(packed_t, features, w1ab, w1c, b1, w2, b2, w3t, b3, *, bm):
    b = features.shape[0]
    n_out = w3t.shape[0]
    const = lambda i: (0, 0)
    return pl.pallas_call(
        _mlp_body,
        grid=(b // bm,),
        in_specs=[
            pl.BlockSpec((PACK_DIM, bm), lambda i: (0, i)),
            pl.BlockSpec((bm, features.shape[1]), lambda i: (i, 0)),
            pl.BlockSpec(w1ab.shape, const),
            pl.BlockSpec(w1c.shape, const),
            pl.BlockSpec(b1.shape, const),
            pl.BlockSpec(w2.shape, const),
            pl.BlockSpec(b2.shape, const),
            pl.BlockSpec(w3t.shape, const),
            pl.BlockSpec(b3.shape, const),
        ],
        out_specs=pl.BlockSpec((n_out, bm), lambda i: (0, i)),
        out_shape=jax.ShapeDtypeStruct((n_out, b), jnp.float32),
        compiler_params=pltpu.CompilerParams(
            dimension_semantics=("parallel",)),
    )(packed_t, features, w1ab, w1c, b1, w2, b2, w3t, b3)


def kernel(port_idx, protocol_idx, features, port_table, proto_table,
           W1, b1, W2, b2, W3, b3):
    b = port_idx.shape[0]
    info = plsc.get_sparse_core_info()
    nc, ns = info.num_cores, info.num_subcores

    table4 = port_table.T.reshape(4, 8, 512, 128).transpose(0, 2, 1, 3)
    packed_t = _sc_gather_t(table4, port_idx.astype(jnp.int32),
                            proto_table.T, protocol_idx.astype(jnp.int32),
                            nc=nc, ns=ns, b=b)

    out_t = _mlp(packed_t, features.astype(jnp.bfloat16),
                 W1[:PACK_DIM], W1[PACK_DIM:].astype(jnp.bfloat16),
                 b1.reshape(1, -1), W2, b2.reshape(1, -1),
                 W3.T, b3.reshape(-1, 1), bm=4096)
    return out_t.T
